# baseline (speedup 1.0000x reference)
import functools

import jax
import jax.numpy as jnp
from jax import lax
from jax.experimental import pallas as pl
from jax.experimental.pallas import tpu as pltpu
from jax.experimental.pallas import tpu_sc as plsc

_CTX = 200
_DIM = 64
_HID = 128
_LANES = 16
_TOK_PER_TILE = 8
_N_TITLE_TILES = _CTX // _TOK_PER_TILE
_NROW_CHUNKS = _DIM // _LANES
_KCHUNK = 1280
_NK = _CTX * _DIM // _KCHUNK


def _gather_body(idx_hbm, embT_hbm, x_hbm, idx_v, win_v, xflat_v, sem,
                 *, num_cores):
    wid = lax.axis_index("s") * num_cores + lax.axis_index("c")

    @pl.when(wid < _N_TITLE_TILES)
    def _():
        base = wid * (_TOK_PER_TILE * _DIM)
        pltpu.sync_copy(idx_hbm.at[pl.ds(wid * _TOK_PER_TILE, _LANES)], idx_v)
        idx = idx_v[...]
        copies = []
        for j in range(_TOK_PER_TILE):
            colb = pl.multiple_of((idx[j] // 128) * 128, 128)
            copies.append(
                pltpu.async_copy(embT_hbm.at[:, pl.ds(colb, 128)],
                                 win_v.at[j], sem))
        rows0 = lax.iota(jnp.int32, _LANES)
        for j in range(_TOK_PER_TILE):
            copies[j].wait()
            off = jnp.full((_LANES,), idx[j] % 128, jnp.int32)
            for b in range(_NROW_CHUNKS):
                vals = plsc.load_gather(win_v.at[j], [rows0 + b * _LANES, off])
                xflat_v[0, pl.ds(j * _DIM + b * _LANES, _LANES)] = vals
        pltpu.sync_copy(
            xflat_v,
            x_hbm.at[:, pl.ds(pl.multiple_of(base, 128), _TOK_PER_TILE * _DIM)])


def _make_sc_gather():
    mesh = plsc.VectorSubcoreMesh(core_axis_name="c", subcore_axis_name="s")
    return pl.kernel(
        functools.partial(_gather_body, num_cores=mesh.num_cores),
        out_type=jax.ShapeDtypeStruct((1, _CTX * _DIM), jnp.float32),
        mesh=mesh,
        compiler_params=pltpu.CompilerParams(disable_bounds_checks=True,
                                             needs_layout_passes=False),
        scratch_types=[
            pltpu.VMEM((_LANES,), jnp.int32),
            pltpu.VMEM((_TOK_PER_TILE, _DIM, 128), jnp.float32),
            pltpu.VMEM((1, _TOK_PER_TILE * _DIM), jnp.float32),
            pltpu.SemaphoreType.DMA,
        ],
    )


def _mlp_body(x_ref, emb2T_ref, cat_ref, q_ref, w1_hbm, b1_ref, w2_ref,
              b2_ref, o_ref, wbuf, wedge, sem, esem):
    ncat = emb2T_ref.shape[1]
    lane = lax.broadcasted_iota(jnp.int32, (_DIM, ncat), 1)
    col = jnp.where(lane == cat_ref[0], emb2T_ref[...], 0.0)
    cat_row = jnp.sum(col, axis=1, keepdims=True)

    edge_lo = pltpu.make_async_copy(w1_hbm.at[pl.ds(0, _DIM)],
                                    wedge.at[pl.ds(0, _DIM)], esem)
    edge_hi = pltpu.make_async_copy(
        w1_hbm.at[pl.ds(_CTX * _DIM + _DIM, 1)],
        wedge.at[pl.ds(_DIM, 1)], esem)
    edge_lo.start()
    edge_hi.start()

    def chunk_copy(i, slot):
        return pltpu.make_async_copy(
            w1_hbm.at[pl.ds(_DIM + i * _KCHUNK, _KCHUNK)], wbuf.at[slot], sem)

    chunk_copy(0, 0).start()
    h = jnp.zeros((1, _HID), jnp.float32)
    for i in range(_NK):
        if i + 1 < _NK:
            chunk_copy(i + 1, (i + 1) % 2).start()
        chunk_copy(i, i % 2).wait()
        h = h + jnp.dot(x_ref[:, pl.ds(i * _KCHUNK, _KCHUNK)],
                        wbuf[i % 2], preferred_element_type=jnp.float32)

    edge_lo.wait()
    edge_hi.wait()
    h = h + jnp.sum(cat_row * wedge[0:_DIM, :], axis=0, keepdims=True)
    h = h + q_ref[...] * wedge[_DIM:_DIM + 1, :]
    h = jnp.maximum(h + b1_ref[...], 0.0)
    o = jnp.dot(h, w2_ref[...], preferred_element_type=jnp.float32)
    o_ref[...] = jax.nn.sigmoid(o + b2_ref[...])


def _mlp(x, emb2T, cat, q, W1, b1, W2, b2):
    vmem = pl.BlockSpec(memory_space=pltpu.MemorySpace.VMEM)
    return pl.pallas_call(
        _mlp_body,
        in_specs=[vmem, vmem,
                  pl.BlockSpec(memory_space=pltpu.MemorySpace.SMEM),
                  vmem,
                  pl.BlockSpec(memory_space=pltpu.MemorySpace.HBM),
                  vmem, vmem, vmem],
        out_shape=jax.ShapeDtypeStruct((1, 1), jnp.float32),
        scratch_shapes=[
            pltpu.VMEM((2, _KCHUNK, _HID), jnp.float32),
            pltpu.VMEM((_DIM + 1, _HID), jnp.float32),
            pltpu.SemaphoreType.DMA,
            pltpu.SemaphoreType.DMA,
        ],
    )(x, emb2T, cat, q, W1, b1, W2, b2)


def kernel(category, title, quantity, emb, emb2, W1, b1, W2, b2):
    x = _make_sc_gather()(title.astype(jnp.int32), emb.T)
    return _mlp(x, emb2.T, category.astype(jnp.int32),
                quantity.reshape(1, 1), W1, b1.reshape(1, _HID),
                W2, b2.reshape(1, 1))


# speedup vs baseline: 1.1387x; 1.1387x over previous
import functools

import jax
import jax.numpy as jnp
from jax import lax
from jax.experimental import pallas as pl
from jax.experimental.pallas import tpu as pltpu
from jax.experimental.pallas import tpu_sc as plsc

_CTX = 200
_DIM = 64
_HID = 128
_LANES = 16
_TOK_PER_TILE = 8
_N_TITLE_TILES = _CTX // _TOK_PER_TILE
_NROW_CHUNKS = _DIM // _LANES
_KCHUNK = 1280
_NK = _CTX * _DIM // _KCHUNK


def _gather_body(idx_hbm, embT_hbm, x_hbm, idx_v, win_v, xflat_v, sem,
                 *, num_cores):
    wid = lax.axis_index("s") * num_cores + lax.axis_index("c")

    @pl.when(wid < _N_TITLE_TILES)
    def _():
        base = wid * (_TOK_PER_TILE * _DIM)
        pltpu.sync_copy(idx_hbm.at[pl.ds(wid * _TOK_PER_TILE, _LANES)], idx_v)
        idx = idx_v[...]
        copies = []
        for j in range(_TOK_PER_TILE):
            colb = pl.multiple_of((idx[j] // 128) * 128, 128)
            copies.append(
                pltpu.async_copy(embT_hbm.at[:, pl.ds(colb, 128)],
                                 win_v.at[j], sem))
        rows0 = lax.iota(jnp.int32, _LANES)
        for j in range(_TOK_PER_TILE):
            copies[j].wait()
            off = jnp.full((_LANES,), idx[j] % 128, jnp.int32)
            for b in range(_NROW_CHUNKS):
                vals = plsc.load_gather(win_v.at[j], [rows0 + b * _LANES, off])
                xflat_v[0, pl.ds(j * _DIM + b * _LANES, _LANES)] = vals
        pltpu.sync_copy(
            xflat_v,
            x_hbm.at[:, pl.ds(pl.multiple_of(base, 128), _TOK_PER_TILE * _DIM)])


def _make_sc_gather():
    mesh = plsc.VectorSubcoreMesh(core_axis_name="c", subcore_axis_name="s")
    return pl.kernel(
        functools.partial(_gather_body, num_cores=mesh.num_cores),
        out_type=jax.ShapeDtypeStruct((1, _CTX * _DIM), jnp.float32),
        mesh=mesh,
        compiler_params=pltpu.CompilerParams(disable_bounds_checks=True,
                                             needs_layout_passes=False),
        scratch_types=[
            pltpu.VMEM((_LANES,), jnp.int32),
            pltpu.VMEM((_TOK_PER_TILE, _DIM, 128), jnp.float32),
            pltpu.VMEM((1, _TOK_PER_TILE * _DIM), jnp.float32),
            pltpu.SemaphoreType.DMA,
        ],
    )


def _mlp_body(x_ref, emb2T_ref, cat_ref, q_ref, w1_ref, b1_ref, w2_ref,
              b2_ref, o_ref):
    ncat = emb2T_ref.shape[1]
    lane = lax.broadcasted_iota(jnp.int32, (_DIM, ncat), 1)
    col = jnp.where(lane == cat_ref[0], emb2T_ref[...], 0.0)
    cat_row = jnp.sum(col, axis=1, keepdims=True)
    h = jnp.sum(cat_row * w1_ref[0:_DIM, :], axis=0, keepdims=True)
    h = h + jnp.dot(x_ref[...], w1_ref[_DIM:_DIM + _CTX * _DIM, :],
                    preferred_element_type=jnp.float32)
    h = h + q_ref[...] * w1_ref[_CTX * _DIM + _DIM:_CTX * _DIM + _DIM + 1, :]
    h = jnp.maximum(h + b1_ref[...], 0.0)
    o = jnp.dot(h, w2_ref[...], preferred_element_type=jnp.float32)
    o_ref[...] = jax.nn.sigmoid(o + b2_ref[...])


def _mlp(x, emb2T, cat, q, W1, b1, W2, b2):
    vmem = pl.BlockSpec(memory_space=pltpu.MemorySpace.VMEM)
    return pl.pallas_call(
        _mlp_body,
        in_specs=[vmem, vmem,
                  pl.BlockSpec(memory_space=pltpu.MemorySpace.SMEM),
                  vmem, vmem, vmem, vmem, vmem],
        out_shape=jax.ShapeDtypeStruct((1, 1), jnp.float32),
    )(x, emb2T, cat, q, W1, b1, W2, b2)


def kernel(category, title, quantity, emb, emb2, W1, b1, W2, b2):
    x = _make_sc_gather()(title.astype(jnp.int32), emb.T)
    return _mlp(x, emb2.T, category.astype(jnp.int32),
                quantity.reshape(1, 1), W1, b1.reshape(1, _HID),
                W2, b2.reshape(1, 1))
